# baseline (speedup 1.0000x reference)
import jax
import jax.numpy as jnp
from jax import lax
from jax.experimental import pallas as pl
from jax.experimental.pallas import tpu as pltpu
from jax.experimental.pallas import tpu_sc as plsc

N_NODES_K = 10000
N_EDGES_K = 320000
D_K = 128
NUM_WORKERS = 32
E_PER_W = N_EDGES_K // NUM_WORKERS
CHUNK = 80
GROUPS = CHUNK // 16
N_CHUNKS = E_PER_W // CHUNK


def _sc_body(z_src, z_dst, src_idx, dst_idx, out,
             idx_a, idx_b, rows_a0, rows_b0, rows_a1, rows_b1,
             rows_a2, rows_b2, out_all,
             sem_a0, sem_b0, sem_a1, sem_b1, sem_a2, sem_b2):
    wid = lax.axis_index("s") * 2 + lax.axis_index("c")
    lane15 = lax.iota(jnp.int32, 16) == 15

    pltpu.sync_copy(src_idx.at[wid], idx_a)
    pltpu.sync_copy(dst_idx.at[wid], idx_b)

    rows = ((rows_a0, rows_b0, sem_a0, sem_b0),
            (rows_a1, rows_b1, sem_a1, sem_b1),
            (rows_a2, rows_b2, sem_a2, sem_b2))

    def issue(c, buf):
        ra, rb, sa, sb = buf
        cp_a = pltpu.async_copy(z_src.at[idx_a.at[c]], ra, sa)
        cp_b = pltpu.async_copy(z_dst.at[idx_b.at[c]], rb, sb)
        return cp_a, cp_b

    def wait(buf):
        ra, rb, sa, sb = buf
        pltpu.make_async_copy(z_src.at[idx_a.at[0]], ra, sa).wait()
        pltpu.make_async_copy(z_dst.at[idx_b.at[0]], rb, sb).wait()

    def compute(c, buf):
        ra, rb, _, _ = buf
        for g in range(GROUPS):
            base_e = g * 16
            accs = [jnp.zeros((16,), jnp.float32) for _ in range(16)]
            for k in range(D_K // 32):
                for j in range(16):
                    e = base_e + j
                    va = plsc.bitcast(ra[e, pl.ds(k * 16, 16)], jnp.bfloat16)
                    vb = plsc.bitcast(rb[e, pl.ds(k * 16, 16)], jnp.bfloat16)
                    p0, p1 = plsc.unpack(va * vb,
                                         format=plsc.PackFormat.INTERLEAVED)
                    accs[j] = accs[j] + (p0 + p1)
            cbase = jnp.full((16,), c * CHUNK + base_e, jnp.int32)
            for j in range(16):
                cs = plsc.cumsum(accs[j])
                plsc.store_scatter(out_all, [cbase + j], cs, mask=lane15)

    issue(0, rows[0])
    issue(1, rows[1])

    def trio_body(i, carry):
        c0 = i * 3
        wait(rows[0])
        issue(c0 + 2, rows[2])
        compute(c0, rows[0])
        wait(rows[1])
        issue(c0 + 3, rows[0])
        compute(c0 + 1, rows[1])
        wait(rows[2])
        issue(c0 + 4, rows[1])
        compute(c0 + 2, rows[2])
        return carry

    lax.fori_loop(0, (N_CHUNKS - 2) // 3, trio_body, 0)
    wait(rows[0])
    compute(N_CHUNKS - 2, rows[0])
    wait(rows[1])
    compute(N_CHUNKS - 1, rows[1])

    pltpu.sync_copy(out_all, out.at[pl.ds(wid * E_PER_W, E_PER_W)])


def kernel(z_src, z_dst, edge_index):
    src_idx = edge_index[0].astype(jnp.int32).reshape(NUM_WORKERS, N_CHUNKS, CHUNK)
    dst_idx = edge_index[1].astype(jnp.int32).reshape(NUM_WORKERS, N_CHUNKS, CHUNK)
    z_src = jax.lax.bitcast_convert_type(
        z_src.astype(jnp.bfloat16).reshape(N_NODES_K, D_K // 2, 2), jnp.int32)
    z_dst = jax.lax.bitcast_convert_type(
        z_dst.astype(jnp.bfloat16).reshape(N_NODES_K, D_K // 2, 2), jnp.int32)
    mesh = plsc.VectorSubcoreMesh(
        core_axis_name="c", subcore_axis_name="s",
        num_cores=2, num_subcores=16)
    kfn = pl.kernel(
        _sc_body,
        out_type=jax.ShapeDtypeStruct((N_EDGES_K,), jnp.float32),
        mesh=mesh,
        compiler_params=pltpu.CompilerParams(
            needs_layout_passes=False, use_tc_tiling_on_sc=False),
        scratch_types=[
            pltpu.VMEM((N_CHUNKS, CHUNK), jnp.int32),
            pltpu.VMEM((N_CHUNKS, CHUNK), jnp.int32),
            pltpu.VMEM((CHUNK, D_K // 2), jnp.int32),
            pltpu.VMEM((CHUNK, D_K // 2), jnp.int32),
            pltpu.VMEM((CHUNK, D_K // 2), jnp.int32),
            pltpu.VMEM((CHUNK, D_K // 2), jnp.int32),
            pltpu.VMEM((CHUNK, D_K // 2), jnp.int32),
            pltpu.VMEM((CHUNK, D_K // 2), jnp.int32),
            pltpu.VMEM((E_PER_W,), jnp.float32),
            pltpu.SemaphoreType.DMA,
            pltpu.SemaphoreType.DMA,
            pltpu.SemaphoreType.DMA,
            pltpu.SemaphoreType.DMA,
            pltpu.SemaphoreType.DMA,
            pltpu.SemaphoreType.DMA,
        ],
    )
    return kfn(z_src, z_dst, src_idx, dst_idx)


# speedup vs baseline: 1.0152x; 1.0152x over previous
import jax
import jax.numpy as jnp
from jax import lax
from jax.experimental import pallas as pl
from jax.experimental.pallas import tpu as pltpu
from jax.experimental.pallas import tpu_sc as plsc

N_NODES_K = 10000
N_EDGES_K = 320000
D_K = 128
NUM_WORKERS = 32
E_PER_W = N_EDGES_K // NUM_WORKERS
CHUNK = 80
GROUPS = CHUNK // 16
N_CHUNKS = E_PER_W // CHUNK


def _sc_body(z_src, z_dst, src_idx, dst_idx, out,
             idx_a, idx_b, rows_a0, rows_b0, rows_a1, rows_b1,
             out_all,
             sem_a0, sem_b0, sem_a1, sem_b1):
    wid = lax.axis_index("s") * 2 + lax.axis_index("c")
    lane15 = lax.iota(jnp.int32, 16) == 15

    pltpu.sync_copy(src_idx.at[wid], idx_a)
    pltpu.sync_copy(dst_idx.at[wid], idx_b)

    rows = ((rows_a0, rows_b0, sem_a0, sem_b0),
            (rows_a1, rows_b1, sem_a1, sem_b1))

    def issue(c, buf):
        ra, rb, sa, sb = buf
        cp_a = pltpu.async_copy(z_src.at[idx_a.at[c]], ra, sa)
        cp_b = pltpu.async_copy(z_dst.at[idx_b.at[c]], rb, sb)
        return cp_a, cp_b

    def wait(buf):
        ra, rb, sa, sb = buf
        pltpu.make_async_copy(z_src.at[idx_a.at[0]], ra, sa).wait()
        pltpu.make_async_copy(z_dst.at[idx_b.at[0]], rb, sb).wait()

    def compute(c, buf):
        ra, rb, _, _ = buf
        for g in range(GROUPS):
            base_e = g * 16
            accs = [jnp.zeros((16,), jnp.float32) for _ in range(16)]
            for k in range(D_K // 32):
                for j in range(16):
                    e = base_e + j
                    va = plsc.bitcast(ra[e, pl.ds(k * 16, 16)], jnp.bfloat16)
                    vb = plsc.bitcast(rb[e, pl.ds(k * 16, 16)], jnp.bfloat16)
                    p0, p1 = plsc.unpack(va * vb,
                                         format=plsc.PackFormat.INTERLEAVED)
                    accs[j] = accs[j] + (p0 + p1)
            cbase = jnp.full((16,), c * CHUNK + base_e, jnp.int32)
            for j in range(16):
                cs = plsc.cumsum(accs[j])
                plsc.store_scatter(out_all, [cbase + j], cs, mask=lane15)

    issue(0, rows[0])

    def pair_body(i, carry):
        c0 = i * 2
        wait(rows[0])
        issue(c0 + 1, rows[1])
        compute(c0, rows[0])
        wait(rows[1])
        issue(c0 + 2, rows[0])
        compute(c0 + 1, rows[1])
        return carry

    lax.fori_loop(0, (N_CHUNKS - 1) // 2, pair_body, 0)
    wait(rows[0])
    compute(N_CHUNKS - 1, rows[0])

    pltpu.sync_copy(out_all, out.at[pl.ds(wid * E_PER_W, E_PER_W)])


def kernel(z_src, z_dst, edge_index):
    src_idx = edge_index[0].astype(jnp.int32).reshape(NUM_WORKERS, N_CHUNKS, CHUNK)
    dst_idx = edge_index[1].astype(jnp.int32).reshape(NUM_WORKERS, N_CHUNKS, CHUNK)
    z_src = jax.lax.bitcast_convert_type(
        z_src.astype(jnp.bfloat16).reshape(N_NODES_K, D_K // 2, 2), jnp.int32)
    z_dst = jax.lax.bitcast_convert_type(
        z_dst.astype(jnp.bfloat16).reshape(N_NODES_K, D_K // 2, 2), jnp.int32)
    mesh = plsc.VectorSubcoreMesh(
        core_axis_name="c", subcore_axis_name="s",
        num_cores=2, num_subcores=16)
    kfn = pl.kernel(
        _sc_body,
        out_type=jax.ShapeDtypeStruct((N_EDGES_K,), jnp.float32),
        mesh=mesh,
        compiler_params=pltpu.CompilerParams(
            needs_layout_passes=False, use_tc_tiling_on_sc=False),
        scratch_types=[
            pltpu.VMEM((N_CHUNKS, CHUNK), jnp.int32),
            pltpu.VMEM((N_CHUNKS, CHUNK), jnp.int32),
            pltpu.VMEM((CHUNK, D_K // 2), jnp.int32),
            pltpu.VMEM((CHUNK, D_K // 2), jnp.int32),
            pltpu.VMEM((CHUNK, D_K // 2), jnp.int32),
            pltpu.VMEM((CHUNK, D_K // 2), jnp.int32),
            pltpu.VMEM((E_PER_W,), jnp.float32),
            pltpu.SemaphoreType.DMA,
            pltpu.SemaphoreType.DMA,
            pltpu.SemaphoreType.DMA,
            pltpu.SemaphoreType.DMA,
        ],
    )
    return kfn(z_src, z_dst, src_idx, dst_idx)
